# baseline (speedup 1.0000x reference)
import functools

import numpy as np
import jax
import jax.numpy as jnp
from jax import lax
from jax.experimental import pallas as pl
from jax.experimental.pallas import tpu as pltpu
from jax.experimental.pallas import tpu_sc as plsc

B = 16384
K = 50
NUM_ATOMS = 1000000

NW = 32
R = B // NW
H = R // 2
E = R * K
EH = H * K
NBLK = H // 8
NVEC = 8 * K // 16


def _scale_body(h_ref, rns_ref, out_ref):
    s = jnp.sum(h_ref[:])
    novelty = jnp.sqrt(jnp.maximum(rns_ref[:], 0.0))
    out_ref[:] = novelty / jnp.maximum(s, 1e-9)


def _tc_scale(h, rns):
    out = pl.pallas_call(
        _scale_body,
        out_shape=jax.ShapeDtypeStruct((128, 128), jnp.float32),
    )(h.reshape(1000, 1000), rns.reshape(128, 128))
    return out.reshape(B)


_mesh = plsc.VectorSubcoreMesh(core_axis_name="c", subcore_axis_name="s")


@functools.partial(
    pl.kernel,
    mesh=_mesh,
    out_type=jax.ShapeDtypeStruct((B,), jnp.float32),
    compiler_params=pltpu.CompilerParams(needs_layout_passes=False),
    scratch_types=[
        pltpu.VMEM((H, K), jnp.int32),
        pltpu.VMEM((H, K), jnp.float32),
        pltpu.VMEM((E,), jnp.int32),
        pltpu.VMEM((E,), jnp.float32),
        pltpu.VMEM((R,), jnp.float32),
        pltpu.VMEM((R,), jnp.float32),
        pltpu.SemaphoreType.DMA,
        pltpu.SemaphoreType.DMA,
    ],
)
def _sc_gather_reduce(idx_hbm, vals_hbm, table_hbm, scale_hbm, out_hbm,
                      idx2d, vals2d, idxf, hf, scale_v, out_v, sem_g, sem_l):
    wid = lax.axis_index("s") * 2 + lax.axis_index("c")
    base_r = wid * R

    iota16 = lax.iota(jnp.int32, 16)

    def pack_half(hh):
        def g_body(g, _):
            rows = iota16 + g * 16
            fbase = hh * EH + g * (16 * K)
            for k in range(K):
                v = plsc.load_gather(idx2d, [rows, jnp.full((16,), k, jnp.int32)])
                idxf[pl.ds(fbase + 16 * k, 16)] = v
            return 0
        lax.fori_loop(0, H // 16, g_body, 0)

    def compute_half(hh):
        def g_body(g, _):
            rows = iota16 + g * 16
            fbase = hh * EH + g * (16 * K)
            acc = jnp.zeros((16,), jnp.float32)
            for k in range(K):
                h16 = hf[pl.ds(fbase + 16 * k, 16)]
                v16 = plsc.load_gather(
                    vals2d, [rows, jnp.full((16,), k, jnp.int32)]
                )
                acc = acc + jnp.maximum(v16, 0.0) * h16
            s16 = scale_v[pl.ds(hh * H + g * 16, 16)]
            out_v[pl.ds(hh * H + g * 16, 16)] = acc * s16
            return 0
        lax.fori_loop(0, H // 16, g_body, 0)

    pltpu.sync_copy(idx_hbm.at[pl.ds(base_r, H), :], idx2d)
    pack_half(0)
    g0 = pltpu.async_copy(table_hbm.at[idxf.at[pl.ds(0, EH)]],
                          hf.at[pl.ds(0, EH)], sem_g)
    pltpu.sync_copy(idx_hbm.at[pl.ds(base_r + H, H), :], idx2d)
    pack_half(1)
    g1 = pltpu.async_copy(table_hbm.at[idxf.at[pl.ds(EH, EH)]],
                          hf.at[pl.ds(EH, EH)], sem_g)
    pltpu.sync_copy(vals_hbm.at[pl.ds(base_r, H), :], vals2d)
    pltpu.sync_copy(scale_hbm.at[pl.ds(base_r, R)], scale_v)
    g0.wait()
    compute_half(0)
    pltpu.sync_copy(vals_hbm.at[pl.ds(base_r + H, H), :], vals2d)
    g1.wait()
    compute_half(1)

    pltpu.sync_copy(out_v, out_hbm.at[pl.ds(base_r, R)])


def kernel(residual_norm_sq, topk_idx, topk_vals, atom_hazard_prior):
    idx = topk_idx.astype(jnp.int32)
    scale = _tc_scale(atom_hazard_prior, residual_norm_sq)
    return _sc_gather_reduce(idx, topk_vals, atom_hazard_prior, scale)


# speedup vs baseline: 1.2691x; 1.2691x over previous
import functools

import numpy as np
import jax
import jax.numpy as jnp
from jax import lax
from jax.experimental import pallas as pl
from jax.experimental.pallas import tpu as pltpu
from jax.experimental.pallas import tpu_sc as plsc

B = 16384
K = 50
NUM_ATOMS = 1000000

NW = 32
R = B // NW
NQ = 4
Q = R // NQ
EQ = Q * K
NBLKQ = Q // 8
NVEC = 8 * K // 16


def _finish_body(h_ref, rns_ref, rowsum_ref, out_ref):
    s = jnp.sum(h_ref[:])
    novelty = jnp.sqrt(jnp.maximum(rns_ref[:], 0.0))
    out_ref[:] = rowsum_ref[:] * novelty / jnp.maximum(s, 1e-9)


def _tc_finish(h, rns, rowsum):
    out = pl.pallas_call(
        _finish_body,
        out_shape=jax.ShapeDtypeStruct((128, 128), jnp.float32),
    )(h.reshape(1000, 1000), rns.reshape(128, 128), rowsum.reshape(128, 128))
    return out.reshape(B)


_mesh = plsc.VectorSubcoreMesh(core_axis_name="c", subcore_axis_name="s")


@functools.partial(
    pl.kernel,
    mesh=_mesh,
    out_type=jax.ShapeDtypeStruct((B,), jnp.float32),
    compiler_params=pltpu.CompilerParams(needs_layout_passes=False),
    scratch_types=[
        pltpu.VMEM((Q, K), jnp.int32),
        pltpu.VMEM((Q, K), jnp.int32),
        pltpu.VMEM((Q, K), jnp.float32),
        pltpu.VMEM((Q, K), jnp.float32),
        pltpu.VMEM((EQ,), jnp.int32),
        pltpu.VMEM((EQ,), jnp.int32),
        pltpu.VMEM((EQ,), jnp.float32),
        pltpu.VMEM((EQ,), jnp.float32),
        pltpu.VMEM((R,), jnp.float32),
        pltpu.SemaphoreType.DMA,
        pltpu.SemaphoreType.DMA,
        pltpu.SemaphoreType.DMA,
        pltpu.SemaphoreType.DMA,
        pltpu.SemaphoreType.DMA,
        pltpu.SemaphoreType.DMA,
    ],
)
def _sc_gather_reduce(idx_hbm, vals_hbm, table_hbm, out_hbm,
                      idx_a, idx_b, vals_a, vals_b, idxf_a, idxf_b,
                      hf_a, hf_b, out_v,
                      sem_i0, sem_i1, sem_v0, sem_v1, sem_g0, sem_g1):
    wid = lax.axis_index("s") * 2 + lax.axis_index("c")
    base_r = wid * R

    idx_bufs = [idx_a, idx_b]
    vals_bufs = [vals_a, vals_b]
    idxf_bufs = [idxf_a, idxf_b]
    hf_bufs = [hf_a, hf_b]
    sem_i = [sem_i0, sem_i1]
    sem_v = [sem_v0, sem_v1]
    sem_g = [sem_g0, sem_g1]

    iota16 = lax.iota(jnp.int32, 16)
    rows_c = [(iota16 + 16 * i) // K for i in range(NVEC)]
    cols_c = [(iota16 + 16 * i) % K for i in range(NVEC)]

    def copy_idx(q, p):
        return pltpu.async_copy(
            idx_hbm.at[pl.ds(base_r + q * Q, Q), :], idx_bufs[p], sem_i[p])

    def copy_vals(q, p):
        return pltpu.async_copy(
            vals_hbm.at[pl.ds(base_r + q * Q, Q), :], vals_bufs[p], sem_v[p])

    def pack_quarter(p):
        idx2d = idx_bufs[p]
        idxf = idxf_bufs[p]
        def blk_body(blk, _):
            fbase = blk * (8 * K)
            for i in range(NVEC):
                r = rows_c[i] + blk * 8
                v = plsc.load_gather(idx2d, [r, cols_c[i]])
                idxf[pl.ds(fbase + 16 * i, 16)] = v
            return 0
        lax.fori_loop(0, NBLKQ, blk_body, 0)

    def compute_quarter(q, p):
        vals2d = vals_bufs[p]
        hf = hf_bufs[p]
        def g_body(g, _):
            rows = lax.iota(jnp.int32, 16) + g * 16
            fbase = g * 16 * K
            acc = jnp.zeros((16,), jnp.float32)
            for k in range(K):
                iv = lax.iota(jnp.int32, 16) * K + (fbase + k)
                h16 = plsc.load_gather(hf, [iv])
                v16 = plsc.load_gather(
                    vals2d, [rows, jnp.full((16,), k, jnp.int32)]
                )
                acc = acc + jnp.maximum(v16, 0.0) * h16
            out_v[pl.ds(q * Q + g * 16, 16)] = acc
            return 0
        lax.fori_loop(0, Q // 16, g_body, 0)

    def fire_gather(p):
        return pltpu.async_copy(
            table_hbm.at[idxf_bufs[p].at[:]], hf_bufs[p], sem_g[p])

    cis = [None, None]
    cvs = [None, None]
    gathers = [None, None]
    cis[0] = copy_idx(0, 0)
    cvs[0] = copy_vals(0, 0)
    cis[0].wait()
    cis[1] = copy_idx(1, 1)
    cvs[1] = copy_vals(1, 1)
    pack_quarter(0)
    gathers[0] = fire_gather(0)
    for q in range(NQ):
        p = q & 1
        if q + 1 < NQ:
            cis[p ^ 1].wait()
            pack_quarter(p ^ 1)
            gathers[p ^ 1] = fire_gather(p ^ 1)
        if q + 2 < NQ:
            cis[p] = copy_idx(q + 2, p)
        cvs[p].wait()
        gathers[p].wait()
        compute_quarter(q, p)
        if q + 2 < NQ:
            cvs[p] = copy_vals(q + 2, p)

    pltpu.sync_copy(out_v, out_hbm.at[pl.ds(base_r, R)])


def kernel(residual_norm_sq, topk_idx, topk_vals, atom_hazard_prior):
    idx = topk_idx.astype(jnp.int32)
    rowsum = _sc_gather_reduce(idx, topk_vals, atom_hazard_prior)
    return _tc_finish(atom_hazard_prior, residual_norm_sq, rowsum)
